# Initial kernel scaffold; baseline (speedup 1.0000x reference)
import functools

import jax
import jax.numpy as jnp
from jax import lax
from jax.experimental import pallas as pl
from jax.experimental.pallas import tpu as pltpu
from jax.experimental.pallas import tpu_sc as plsc

N = 10000
E = 320000
D = 128

NC = 2
NS = 16
NW = NC * NS
CHUNK = 128
NCHUNK = 80
EPW = CHUNK * NCHUNK
EP = EPW * NW
NP = 10240
RPT = NP // NS

_mesh = plsc.VectorSubcoreMesh(core_axis_name="c", subcore_axis_name="s")


@functools.partial(
    pl.kernel,
    out_type=jax.ShapeDtypeStruct((NC, NP, 16), jnp.float32),
    mesh=_mesh,
    scratch_types=[
        pltpu.VMEM((NCHUNK, CHUNK), jnp.int32),
        pltpu.VMEM((CHUNK, 16), jnp.float32),
        pltpu.VMEM((RPT, 16), jnp.float32),
        pltpu.VMEM_SHARED((NP, 16), jnp.float32),
    ],
)
def _sc_degree(dst_hbm, ones_hbm, zeros_hbm, out_hbm, dstv, onesv, zv, deg_sh):
    c = lax.axis_index("c")
    s = lax.axis_index("s")
    w = c * NS + s
    pltpu.sync_copy(zeros_hbm, zv)
    pltpu.sync_copy(zv, deg_sh.at[pl.ds(s * RPT, RPT)])
    pltpu.sync_copy(ones_hbm, onesv)
    pltpu.sync_copy(dst_hbm.at[w], dstv)
    plsc.subcore_barrier()

    @pl.loop(0, NCHUNK)
    def _(j):
        pltpu.sync_copy(onesv, deg_sh.at[dstv.at[j]], add=True)

    plsc.subcore_barrier()
    pltpu.sync_copy(deg_sh.at[pl.ds(s * RPT, RPT)],
                    out_hbm.at[c, pl.ds(s * RPT, RPT)])


@functools.partial(
    pl.kernel,
    out_type=jax.ShapeDtypeStruct((NC, NP, D), jnp.float32),
    mesh=_mesh,
    scratch_types=[
        pltpu.VMEM((NCHUNK, CHUNK), jnp.int32),
        pltpu.VMEM((NCHUNK, CHUNK), jnp.int32),
        pltpu.VMEM((CHUNK, D), jnp.float32),
        pltpu.VMEM((CHUNK, D), jnp.float32),
        pltpu.VMEM_SHARED((NP, D), jnp.float32),
    ],
)
def _sc_aggregate(u_hbm, src_hbm, dst_hbm, zeros_hbm, out_hbm,
                  srcv, dstv, rows, zv, acc_sh):
    c = lax.axis_index("c")
    s = lax.axis_index("s")
    w = c * NS + s
    pltpu.sync_copy(zeros_hbm, zv)

    @pl.loop(0, RPT // CHUNK)
    def _(k):
        pltpu.sync_copy(zv, acc_sh.at[pl.ds(s * RPT + k * CHUNK, CHUNK)])

    pltpu.sync_copy(src_hbm.at[w], srcv)
    pltpu.sync_copy(dst_hbm.at[w], dstv)
    plsc.subcore_barrier()

    @pl.loop(0, NCHUNK)
    def _(j):
        pltpu.sync_copy(u_hbm.at[srcv.at[j]], rows)
        pltpu.sync_copy(rows, acc_sh.at[dstv.at[j]], add=True)

    plsc.subcore_barrier()
    pltpu.sync_copy(acc_sh.at[pl.ds(s * RPT, RPT)],
                    out_hbm.at[c, pl.ds(s * RPT, RPT)])


_BLK = 128
_NBLK = NP // _BLK


def _row_spec():
    return pl.BlockSpec((_BLK, D), lambda i: (i, 0))


def _full_spec():
    return pl.BlockSpec((D, D), lambda i: (0, 0))


def _mm_body(x_ref, w_ref, o_ref):
    o_ref[...] = jnp.dot(x_ref[...], w_ref[...],
                         preferred_element_type=jnp.float32)


def _tc_matmul(x, w):
    return pl.pallas_call(
        _mm_body,
        grid=(_NBLK,),
        in_specs=[_row_spec(), _full_spec()],
        out_specs=_row_spec(),
        out_shape=jax.ShapeDtypeStruct((NP, D), jnp.float32),
    )(x, w)


def _scale_body(d0_ref, d1_ref, xw_ref, u_ref, dinv_ref):
    deg = 1.0 + d0_ref[:, 0:1] + d1_ref[:, 0:1]
    dinv = lax.rsqrt(deg)
    dinvb = jnp.broadcast_to(dinv, (_BLK, D))
    dinv_ref[...] = dinvb
    u_ref[...] = dinvb * xw_ref[...]


def _tc_scale(d0, d1, xw):
    deg_spec = pl.BlockSpec((_BLK, 16), lambda i: (i, 0))
    return pl.pallas_call(
        _scale_body,
        grid=(_NBLK,),
        in_specs=[deg_spec, deg_spec, _row_spec()],
        out_specs=[_row_spec(), _row_spec()],
        out_shape=[jax.ShapeDtypeStruct((NP, D), jnp.float32),
                   jax.ShapeDtypeStruct((NP, D), jnp.float32)],
    )(d0, d1, xw)


def _combine_body(a0_ref, a1_ref, u_ref, dinv_ref, b_ref, w_ref, o_ref):
    s = a0_ref[...] + a1_ref[...] + u_ref[...]
    h = jax.nn.relu(dinv_ref[...] * s + b_ref[...])
    o_ref[...] = dinv_ref[...] * jnp.dot(h, w_ref[...],
                                         preferred_element_type=jnp.float32)


def _tc_combine_mm(a0, a1, u, dinvb, b, w):
    bias_spec = pl.BlockSpec((1, D), lambda i: (0, 0))
    return pl.pallas_call(
        _combine_body,
        grid=(_NBLK,),
        in_specs=[_row_spec(), _row_spec(), _row_spec(), _row_spec(),
                  bias_spec, _full_spec()],
        out_specs=_row_spec(),
        out_shape=jax.ShapeDtypeStruct((NP, D), jnp.float32),
    )(a0, a1, u, dinvb, b, w)


def _final_body(a0_ref, a1_ref, u_ref, dinv_ref, b_ref, o_ref):
    s = a0_ref[...] + a1_ref[...] + u_ref[...]
    o_ref[...] = jax.nn.relu(dinv_ref[...] * s + b_ref[...])


def _tc_final(a0, a1, u, dinvb, b):
    bias_spec = pl.BlockSpec((1, D), lambda i: (0, 0))
    return pl.pallas_call(
        _final_body,
        grid=(_NBLK,),
        in_specs=[_row_spec(), _row_spec(), _row_spec(), _row_spec(),
                  bias_spec],
        out_specs=_row_spec(),
        out_shape=jax.ShapeDtypeStruct((NP, D), jnp.float32),
    )(a0, a1, u, dinvb, b)


def kernel(features, adj, W1, b1, W2, b2):
    adj = adj.astype(jnp.int32)
    src = jnp.concatenate([adj[0], jnp.zeros((EP - E,), jnp.int32)])
    dst = jnp.concatenate([adj[1], jnp.full((EP - E,), NP - 1, jnp.int32)])
    src3 = src.reshape(NW, NCHUNK, CHUNK)
    dst3 = dst.reshape(NW, NCHUNK, CHUNK)

    featp = jnp.pad(features, ((0, NP - N), (0, 0)))
    ones16 = jnp.ones((CHUNK, 16), jnp.float32)
    zeros16 = jnp.zeros((RPT, 16), jnp.float32)
    zerosD = jnp.zeros((CHUNK, D), jnp.float32)
    b1r = b1.reshape(1, D)
    b2r = b2.reshape(1, D)

    deg = _sc_degree(dst3, ones16, zeros16)
    xw1 = _tc_matmul(featp, W1)
    u1, dinvb = _tc_scale(deg[0], deg[1], xw1)

    acc1 = _sc_aggregate(u1, src3, dst3, zerosD)
    u2 = _tc_combine_mm(acc1[0], acc1[1], u1, dinvb, b1r, W2)

    acc2 = _sc_aggregate(u2, src3, dst3, zerosD)
    h = _tc_final(acc2[0], acc2[1], u2, dinvb, b2r)

    return h[:N]


# speedup vs baseline: 7.5397x; 7.5397x over previous
import dataclasses
import functools

import jax
import jax.numpy as jnp
from jax import lax
from jax.experimental import pallas as pl
from jax.experimental.pallas import tpu as pltpu
from jax.experimental.pallas import tpu_sc as plsc

N = 10000
E = 320000
D = 128

NC = 2
NS = 16
NW = NC * NS
CHUNK = 128
NCHUNK = 80
EPW = CHUNK * NCHUNK
EP = EPW * NW
NP = 10240
RPT = NP // NS


@functools.cache
def _sc_degree_kernel():
    mesh = plsc.VectorSubcoreMesh(core_axis_name="c", subcore_axis_name="s")
    cp = pltpu.CompilerParams()
    if "needs_layout_passes" in pltpu.CompilerParams.__dataclass_fields__:
        cp = dataclasses.replace(cp, needs_layout_passes=False)
    return pl.kernel(
        _sc_degree_body,
        out_type=jax.ShapeDtypeStruct((NW, NP), jnp.float32),
        mesh=mesh,
        compiler_params=cp,
        scratch_types=[
            pltpu.VMEM((NCHUNK, CHUNK), jnp.int32),
            pltpu.VMEM((NP,), jnp.float32),
        ],
    )


def _sc_degree(dst3):
    return _sc_degree_kernel()(dst3)


def _sc_degree_body(dst_hbm, out_hbm, dstv, degv):
    c = lax.axis_index("c")
    s = lax.axis_index("s")
    w = c * NS + s
    pltpu.sync_copy(dst_hbm.at[w], dstv)

    @pl.loop(0, NP, step=16)
    def _(i):
        degv[pl.ds(i, 16)] = jnp.zeros((16,), jnp.float32)

    ones = jnp.full((16,), 1.0, jnp.float32)

    @pl.loop(0, NCHUNK)
    def _(j):
        for k in range(CHUNK // 16):
            iv = dstv[j, pl.ds(k * 16, 16)]
            plsc.addupdate_scatter(degv, [iv], ones)

    pltpu.sync_copy(degv, out_hbm.at[w])


@functools.cache
def _sc_aggregate_kernel():
    mesh = plsc.VectorSubcoreMesh(core_axis_name="c", subcore_axis_name="s")
    return pl.kernel(
        _sc_aggregate_body,
        out_type=jax.ShapeDtypeStruct((NC, NP, D), jnp.float32),
        mesh=mesh,
        scratch_types=[
            pltpu.VMEM((NCHUNK, CHUNK), jnp.int32),
            pltpu.VMEM((NCHUNK, CHUNK), jnp.int32),
            pltpu.VMEM((CHUNK, D), jnp.float32),
            pltpu.VMEM_SHARED((NP, D), jnp.float32),
        ],
    )


def _sc_aggregate(u, src3, dst3, zerosD):
    return _sc_aggregate_kernel()(u, src3, dst3, zerosD)


def _sc_aggregate_body(u_hbm, src_hbm, dst_hbm, zeros_hbm, out_hbm,
                       srcv, dstv, rows, acc_sh):
    c = lax.axis_index("c")
    s = lax.axis_index("s")
    w = c * NS + s
    pltpu.sync_copy(zeros_hbm, rows)

    @pl.loop(0, RPT // CHUNK)
    def _(k):
        pltpu.sync_copy(rows, acc_sh.at[pl.ds(s * RPT + k * CHUNK, CHUNK)])

    pltpu.sync_copy(src_hbm.at[w], srcv)
    pltpu.sync_copy(dst_hbm.at[w], dstv)
    plsc.subcore_barrier()

    @pl.loop(0, NCHUNK)
    def _(j):
        pltpu.sync_copy(u_hbm.at[srcv.at[j]], rows)
        pltpu.sync_copy(rows, acc_sh.at[dstv.at[j]], add=True)

    plsc.subcore_barrier()
    pltpu.sync_copy(acc_sh.at[pl.ds(s * RPT, RPT)],
                    out_hbm.at[c, pl.ds(s * RPT, RPT)])


_BLK = 128
_NBLK = NP // _BLK


def _row_spec():
    return pl.BlockSpec((_BLK, D), lambda i: (i, 0))


def _full_spec():
    return pl.BlockSpec((D, D), lambda i: (0, 0))


def _mm_body(x_ref, w_ref, o_ref):
    o_ref[...] = jnp.dot(x_ref[...], w_ref[...],
                         preferred_element_type=jnp.float32)


def _tc_matmul(x, w):
    return pl.pallas_call(
        _mm_body,
        grid=(_NBLK,),
        in_specs=[_row_spec(), _full_spec()],
        out_specs=_row_spec(),
        out_shape=jax.ShapeDtypeStruct((NP, D), jnp.float32),
    )(x, w)


def _scale_body(dp_ref, xw_ref, u_ref, dinv_ref):
    ones_w = jnp.ones((NW, 1), jnp.float32)
    colsum = lax.dot_general(dp_ref[...], ones_w, (((0,), (0,)), ((), ())),
                             preferred_element_type=jnp.float32)
    dinv = lax.rsqrt(1.0 + colsum)
    dinvb = jnp.broadcast_to(dinv, (_BLK, D))
    dinv_ref[...] = dinvb
    u_ref[...] = dinvb * xw_ref[...]


def _tc_scale(dp, xw):
    deg_spec = pl.BlockSpec((NW, _BLK), lambda i: (0, i))
    return pl.pallas_call(
        _scale_body,
        grid=(_NBLK,),
        in_specs=[deg_spec, _row_spec()],
        out_specs=[_row_spec(), _row_spec()],
        out_shape=[jax.ShapeDtypeStruct((NP, D), jnp.float32),
                   jax.ShapeDtypeStruct((NP, D), jnp.float32)],
    )(dp, xw)


def _combine_body(a0_ref, a1_ref, u_ref, dinv_ref, b_ref, w_ref, o_ref):
    s = a0_ref[...] + a1_ref[...] + u_ref[...]
    h = jax.nn.relu(dinv_ref[...] * s + b_ref[...])
    o_ref[...] = dinv_ref[...] * jnp.dot(h, w_ref[...],
                                         preferred_element_type=jnp.float32)


def _tc_combine_mm(a0, a1, u, dinvb, b, w):
    bias_spec = pl.BlockSpec((1, D), lambda i: (0, 0))
    return pl.pallas_call(
        _combine_body,
        grid=(_NBLK,),
        in_specs=[_row_spec(), _row_spec(), _row_spec(), _row_spec(),
                  bias_spec, _full_spec()],
        out_specs=_row_spec(),
        out_shape=jax.ShapeDtypeStruct((NP, D), jnp.float32),
    )(a0, a1, u, dinvb, b, w)


def _final_body(a0_ref, a1_ref, u_ref, dinv_ref, b_ref, o_ref):
    s = a0_ref[...] + a1_ref[...] + u_ref[...]
    o_ref[...] = jax.nn.relu(dinv_ref[...] * s + b_ref[...])


def _tc_final(a0, a1, u, dinvb, b):
    bias_spec = pl.BlockSpec((1, D), lambda i: (0, 0))
    return pl.pallas_call(
        _final_body,
        grid=(_NBLK,),
        in_specs=[_row_spec(), _row_spec(), _row_spec(), _row_spec(),
                  bias_spec],
        out_specs=_row_spec(),
        out_shape=jax.ShapeDtypeStruct((NP, D), jnp.float32),
    )(a0, a1, u, dinvb, b)


def kernel(features, adj, W1, b1, W2, b2):
    adj = adj.astype(jnp.int32)
    src = jnp.concatenate([adj[0], jnp.zeros((EP - E,), jnp.int32)])
    dst = jnp.concatenate([adj[1], jnp.full((EP - E,), NP - 1, jnp.int32)])
    src3 = src.reshape(NW, NCHUNK, CHUNK)
    dst3 = dst.reshape(NW, NCHUNK, CHUNK)

    featp = jnp.pad(features, ((0, NP - N), (0, 0)))
    zerosD = jnp.zeros((CHUNK, D), jnp.float32)
    b1r = b1.reshape(1, D)
    b2r = b2.reshape(1, D)

    degp = _sc_degree(dst3)
    xw1 = _tc_matmul(featp, W1)
    u1, dinvb = _tc_scale(degp, xw1)

    acc1 = _sc_aggregate(u1, src3, dst3, zerosD)
    u2 = _tc_combine_mm(acc1[0], acc1[1], u1, dinvb, b1r, W2)

    acc2 = _sc_aggregate(u2, src3, dst3, zerosD)
    h = _tc_final(acc2[0], acc2[1], u2, dinvb, b2r)

    return h[:N]


# speedup vs baseline: 8.4760x; 1.1242x over previous
import dataclasses
import functools

import jax
import jax.numpy as jnp
from jax import lax
from jax.experimental import pallas as pl
from jax.experimental.pallas import tpu as pltpu
from jax.experimental.pallas import tpu_sc as plsc

N = 10000
E = 320000
D = 128

NC = 2
NS = 16
NW = NC * NS
CHUNK = 128
NCHUNK = 80
STRIPE = 16
NSTRIPE = NCHUNK // STRIPE
EPW = CHUNK * NCHUNK
EP = EPW * NW
NCHUNK_A = 128
NCHUNK_B = 32
NP = 10240
RPT = NP // NS


@functools.cache
def _sc_degree_kernel():
    mesh = plsc.VectorSubcoreMesh(core_axis_name="c", subcore_axis_name="s")
    cp = pltpu.CompilerParams()
    if "needs_layout_passes" in pltpu.CompilerParams.__dataclass_fields__:
        cp = dataclasses.replace(cp, needs_layout_passes=False)
    return pl.kernel(
        _sc_degree_body,
        out_type=jax.ShapeDtypeStruct((NW, NP), jnp.float32),
        mesh=mesh,
        compiler_params=cp,
        scratch_types=[
            pltpu.VMEM((NCHUNK, CHUNK), jnp.int32),
            pltpu.VMEM((NP,), jnp.float32),
        ],
    )


def _sc_degree(dst3):
    return _sc_degree_kernel()(dst3)


def _sc_degree_body(dst_hbm, out_hbm, dstv, degv):
    c = lax.axis_index("c")
    s = lax.axis_index("s")
    w = c * NS + s
    pltpu.sync_copy(dst_hbm.at[w], dstv)

    @pl.loop(0, NP, step=16)
    def _(i):
        degv[pl.ds(i, 16)] = jnp.zeros((16,), jnp.float32)

    ones = jnp.full((16,), 1.0, jnp.float32)

    @pl.loop(0, NCHUNK)
    def _(j):
        for k in range(CHUNK // 16):
            iv = dstv[j, pl.ds(k * 16, 16)]
            plsc.addupdate_scatter(degv, [iv], ones)

    pltpu.sync_copy(degv, out_hbm.at[w])


@functools.cache
def _sc_aggregate_kernel():
    mesh = plsc.VectorSubcoreMesh(core_axis_name="c", subcore_axis_name="s")
    return pl.kernel(
        _sc_aggregate_body,
        out_type=jax.ShapeDtypeStruct((NC, NP, D), jnp.float32),
        mesh=mesh,
        scratch_types=[
            pltpu.VMEM((STRIPE, CHUNK), jnp.int32),
            pltpu.VMEM((STRIPE, CHUNK), jnp.int32),
            pltpu.VMEM((CHUNK, D), jnp.float32),
            pltpu.VMEM((CHUNK, D), jnp.float32),
            pltpu.SemaphoreType.DMA,
            pltpu.SemaphoreType.DMA,
            pltpu.VMEM_SHARED((NP, D), jnp.float32),
        ],
    )


def _sc_aggregate(u, srcA, dstA, srcB, dstB, zerosD):
    return _sc_aggregate_kernel()(u, srcA, dstA, srcB, dstB, zerosD)


def _sc_aggregate_body(u_hbm, srcA_hbm, dstA_hbm, srcB_hbm, dstB_hbm,
                       zeros_hbm, out_hbm,
                       srcv, dstv, rows0, rows1, sem0, sem1, acc_sh):
    c = lax.axis_index("c")
    s = lax.axis_index("s")
    pltpu.sync_copy(zeros_hbm, rows0)

    @pl.loop(0, RPT // CHUNK)
    def _(k):
        pltpu.sync_copy(rows0, acc_sh.at[pl.ds(s * RPT + k * CHUNK, CHUNK)])

    plsc.subcore_barrier()

    def stripe(src_hbm, dst_hbm, t):
        pltpu.sync_copy(src_hbm.at[s, pl.ds(t * STRIPE, STRIPE)], srcv)
        pltpu.sync_copy(dst_hbm.at[s, pl.ds(t * STRIPE, STRIPE)], dstv)
        pltpu.async_copy(u_hbm.at[srcv.at[0]], rows0, sem0)

        @pl.loop(0, STRIPE // 2)
        def _(i):
            j = 2 * i
            h1 = pltpu.async_copy(u_hbm.at[srcv.at[j + 1]], rows1, sem1)
            pltpu.make_async_copy(u_hbm.at[srcv.at[j]], rows0, sem0).wait()
            pltpu.sync_copy(rows0, acc_sh.at[dstv.at[j]], add=True)

            @pl.when(j + 2 < STRIPE)
            def _():
                pltpu.async_copy(u_hbm.at[srcv.at[j + 2]], rows0, sem0)

            h1.wait()
            pltpu.sync_copy(rows1, acc_sh.at[dstv.at[j + 1]], add=True)

    @pl.when(c == 0)
    def _():
        for t in range(NCHUNK_A // STRIPE):
            stripe(srcA_hbm, dstA_hbm, t)

    @pl.when(c == 1)
    def _():
        for t in range(NCHUNK_B // STRIPE):
            stripe(srcB_hbm, dstB_hbm, t)

    plsc.subcore_barrier()
    pltpu.sync_copy(acc_sh.at[pl.ds(s * RPT, RPT)],
                    out_hbm.at[c, pl.ds(s * RPT, RPT)])


_BLK = 128
_NBLK = NP // _BLK


def _row_spec():
    return pl.BlockSpec((_BLK, D), lambda i: (i, 0))


def _full_spec():
    return pl.BlockSpec((D, D), lambda i: (0, 0))


def _mm_body(x_ref, w_ref, o_ref):
    o_ref[...] = jnp.dot(x_ref[...], w_ref[...],
                         preferred_element_type=jnp.float32)


def _tc_matmul(x, w):
    return pl.pallas_call(
        _mm_body,
        grid=(_NBLK,),
        in_specs=[_row_spec(), _full_spec()],
        out_specs=_row_spec(),
        out_shape=jax.ShapeDtypeStruct((NP, D), jnp.float32),
    )(x, w)


def _scale_body(dp_ref, xw_ref, u_ref, dinv_ref):
    ones_w = jnp.ones((NW, 1), jnp.float32)
    colsum = lax.dot_general(dp_ref[...], ones_w, (((0,), (0,)), ((), ())),
                             preferred_element_type=jnp.float32)
    dinv = lax.rsqrt(1.0 + colsum)
    dinvb = jnp.broadcast_to(dinv, (_BLK, D))
    dinv_ref[...] = dinvb
    u_ref[...] = dinvb * xw_ref[...]


def _tc_scale(dp, xw):
    deg_spec = pl.BlockSpec((NW, _BLK), lambda i: (0, i))
    return pl.pallas_call(
        _scale_body,
        grid=(_NBLK,),
        in_specs=[deg_spec, _row_spec()],
        out_specs=[_row_spec(), _row_spec()],
        out_shape=[jax.ShapeDtypeStruct((NP, D), jnp.float32),
                   jax.ShapeDtypeStruct((NP, D), jnp.float32)],
    )(dp, xw)


def _combine_body(a0_ref, a1_ref, u_ref, dinv_ref, b_ref, w_ref, o_ref):
    s = a0_ref[...] + a1_ref[...] + u_ref[...]
    h = jax.nn.relu(dinv_ref[...] * s + b_ref[...])
    o_ref[...] = dinv_ref[...] * jnp.dot(h, w_ref[...],
                                         preferred_element_type=jnp.float32)


def _tc_combine_mm(a0, a1, u, dinvb, b, w):
    bias_spec = pl.BlockSpec((1, D), lambda i: (0, 0))
    return pl.pallas_call(
        _combine_body,
        grid=(_NBLK,),
        in_specs=[_row_spec(), _row_spec(), _row_spec(), _row_spec(),
                  bias_spec, _full_spec()],
        out_specs=_row_spec(),
        out_shape=jax.ShapeDtypeStruct((NP, D), jnp.float32),
    )(a0, a1, u, dinvb, b, w)


def _final_body(a0_ref, a1_ref, u_ref, dinv_ref, b_ref, o_ref):
    s = a0_ref[...] + a1_ref[...] + u_ref[...]
    o_ref[...] = jax.nn.relu(dinv_ref[...] * s + b_ref[...])


def _tc_final(a0, a1, u, dinvb, b):
    bias_spec = pl.BlockSpec((1, D), lambda i: (0, 0))
    return pl.pallas_call(
        _final_body,
        grid=(_NBLK,),
        in_specs=[_row_spec(), _row_spec(), _row_spec(), _row_spec(),
                  bias_spec],
        out_specs=_row_spec(),
        out_shape=jax.ShapeDtypeStruct((NP, D), jnp.float32),
    )(a0, a1, u, dinvb, b)


def kernel(features, adj, W1, b1, W2, b2):
    adj = adj.astype(jnp.int32)
    src = jnp.concatenate([adj[0], jnp.zeros((EP - E,), jnp.int32)])
    dst = jnp.concatenate([adj[1], jnp.full((EP - E,), NP - 1, jnp.int32)])
    src3 = src.reshape(NW, NCHUNK, CHUNK)
    dst3 = dst.reshape(NW, NCHUNK, CHUNK)
    nA = NS * NCHUNK_A * CHUNK
    srcA = src[:nA].reshape(NS, NCHUNK_A, CHUNK)
    dstA = dst[:nA].reshape(NS, NCHUNK_A, CHUNK)
    srcB = src[nA:].reshape(NS, NCHUNK_B, CHUNK)
    dstB = dst[nA:].reshape(NS, NCHUNK_B, CHUNK)

    featp = jnp.pad(features, ((0, NP - N), (0, 0)))
    zerosD = jnp.zeros((CHUNK, D), jnp.float32)
    b1r = b1.reshape(1, D)
    b2r = b2.reshape(1, D)

    degp = _sc_degree(dst3)
    xw1 = _tc_matmul(featp, W1)
    u1, dinvb = _tc_scale(degp, xw1)

    acc1 = _sc_aggregate(u1, srcA, dstA, srcB, dstB, zerosD)
    u2 = _tc_combine_mm(acc1[0], acc1[1], u1, dinvb, b1r, W2)

    acc2 = _sc_aggregate(u2, srcA, dstA, srcB, dstB, zerosD)
    h = _tc_final(acc2[0], acc2[1], u2, dinvb, b2r)

    return h[:N]


# speedup vs baseline: 8.8603x; 1.0453x over previous
import dataclasses
import functools

import jax
import jax.numpy as jnp
from jax import lax
from jax.experimental import pallas as pl
from jax.experimental.pallas import tpu as pltpu
from jax.experimental.pallas import tpu_sc as plsc

N = 10000
E = 320000
D = 128

NC = 2
NS = 16
NW = NC * NS
CHUNK = 128
NCHUNK = 80
STRIPE = 16
NSTRIPE = NCHUNK // STRIPE
EPW = CHUNK * NCHUNK
EP = EPW * NW
NCHUNK_A = 160
NP = 10240
RPT = NP // NS


@functools.cache
def _sc_degree_kernel():
    mesh = plsc.VectorSubcoreMesh(core_axis_name="c", subcore_axis_name="s")
    cp = pltpu.CompilerParams()
    if "needs_layout_passes" in pltpu.CompilerParams.__dataclass_fields__:
        cp = dataclasses.replace(cp, needs_layout_passes=False)
    return pl.kernel(
        _sc_degree_body,
        out_type=jax.ShapeDtypeStruct((NW, NP), jnp.float32),
        mesh=mesh,
        compiler_params=cp,
        scratch_types=[
            pltpu.VMEM((NCHUNK, CHUNK), jnp.int32),
            pltpu.VMEM((NP,), jnp.float32),
        ],
    )


def _sc_degree(dst3):
    return _sc_degree_kernel()(dst3)


def _sc_degree_body(dst_hbm, out_hbm, dstv, degv):
    c = lax.axis_index("c")
    s = lax.axis_index("s")
    w = c * NS + s
    pltpu.sync_copy(dst_hbm.at[w], dstv)

    @pl.loop(0, NP, step=16)
    def _(i):
        degv[pl.ds(i, 16)] = jnp.zeros((16,), jnp.float32)

    ones = jnp.full((16,), 1.0, jnp.float32)

    @pl.loop(0, NCHUNK)
    def _(j):
        for k in range(CHUNK // 16):
            iv = dstv[j, pl.ds(k * 16, 16)]
            plsc.addupdate_scatter(degv, [iv], ones)

    pltpu.sync_copy(degv, out_hbm.at[w])


@functools.cache
def _sc_aggregate_kernel():
    mesh = plsc.VectorSubcoreMesh(core_axis_name="c", subcore_axis_name="s")
    return pl.kernel(
        _sc_aggregate_body,
        out_type=jax.ShapeDtypeStruct((NP, D), jnp.float32),
        mesh=mesh,
        scratch_types=[
            pltpu.VMEM((STRIPE, CHUNK), jnp.int32),
            pltpu.VMEM((STRIPE, CHUNK), jnp.int32),
            pltpu.VMEM((CHUNK, D), jnp.float32),
            pltpu.VMEM((CHUNK, D), jnp.float32),
            pltpu.SemaphoreType.DMA,
            pltpu.SemaphoreType.DMA,
            pltpu.VMEM_SHARED((NP, D), jnp.float32),
        ],
    )


def _sc_aggregate(u, srcA, dstA, zerosD):
    return _sc_aggregate_kernel()(u, srcA, dstA, zerosD)


def _sc_aggregate_body(u_hbm, srcA_hbm, dstA_hbm,
                       zeros_hbm, out_hbm,
                       srcv, dstv, rows0, rows1, sem0, sem1, acc_sh):
    c = lax.axis_index("c")
    s = lax.axis_index("s")

    @pl.when(c == 0)
    def _():
        pltpu.sync_copy(zeros_hbm, rows0)

        @pl.loop(0, RPT // CHUNK)
        def _(k):
            pltpu.sync_copy(rows0, acc_sh.at[pl.ds(s * RPT + k * CHUNK, CHUNK)])

    plsc.subcore_barrier()

    def stripe(src_hbm, dst_hbm, t):
        pltpu.sync_copy(src_hbm.at[s, pl.ds(t * STRIPE, STRIPE)], srcv)
        pltpu.sync_copy(dst_hbm.at[s, pl.ds(t * STRIPE, STRIPE)], dstv)
        pltpu.async_copy(u_hbm.at[srcv.at[0]], rows0, sem0)

        @pl.loop(0, STRIPE // 2)
        def _(i):
            j = 2 * i
            h1 = pltpu.async_copy(u_hbm.at[srcv.at[j + 1]], rows1, sem1)
            pltpu.make_async_copy(u_hbm.at[srcv.at[j]], rows0, sem0).wait()
            pltpu.sync_copy(rows0, acc_sh.at[dstv.at[j]], add=True)

            @pl.when(j + 2 < STRIPE)
            def _():
                pltpu.async_copy(u_hbm.at[srcv.at[j + 2]], rows0, sem0)

            h1.wait()
            pltpu.sync_copy(rows1, acc_sh.at[dstv.at[j + 1]], add=True)

    @pl.when(c == 0)
    def _():
        for t in range(NCHUNK_A // STRIPE):
            stripe(srcA_hbm, dstA_hbm, t)

    plsc.subcore_barrier()

    @pl.when(c == 0)
    def _():
        pltpu.sync_copy(acc_sh.at[pl.ds(s * RPT, RPT)],
                        out_hbm.at[pl.ds(s * RPT, RPT)])


_BLK = 1024
_NBLK = NP // _BLK


def _row_spec():
    return pl.BlockSpec((_BLK, D), lambda i: (i, 0))


def _full_spec():
    return pl.BlockSpec((D, D), lambda i: (0, 0))


def _bias_spec():
    return pl.BlockSpec((1, D), lambda i: (0, 0))


def _mm_scale_body(x_ref, w_ref, dp_ref, u_ref, dinv_ref):
    ones_w = jnp.ones((NW, 1), jnp.float32)
    colsum = lax.dot_general(dp_ref[...], ones_w, (((0,), (0,)), ((), ())),
                             preferred_element_type=jnp.float32)
    dinv = lax.rsqrt(1.0 + colsum)
    dinvb = jnp.broadcast_to(dinv, (_BLK, D))
    dinv_ref[...] = dinvb
    xw = jnp.dot(x_ref[...], w_ref[...], preferred_element_type=jnp.float32)
    u_ref[...] = dinvb * xw


def _tc_mm_scale(x, w, dp):
    deg_spec = pl.BlockSpec((NW, _BLK), lambda i: (0, i))
    return pl.pallas_call(
        _mm_scale_body,
        grid=(_NBLK,),
        in_specs=[_row_spec(), _full_spec(), deg_spec],
        out_specs=[_row_spec(), _row_spec()],
        out_shape=[jax.ShapeDtypeStruct((NP, D), jnp.float32),
                   jax.ShapeDtypeStruct((NP, D), jnp.float32)],
    )(x, w, dp)


def _combine_body(a_ref, u_ref, dinv_ref, b_ref, w_ref, o_ref):
    s = a_ref[...] + u_ref[...]
    h = jax.nn.relu(dinv_ref[...] * s + b_ref[...])
    o_ref[...] = dinv_ref[...] * jnp.dot(h, w_ref[...],
                                         preferred_element_type=jnp.float32)


def _tc_combine_mm(a, u, dinvb, b, w):
    return pl.pallas_call(
        _combine_body,
        grid=(_NBLK,),
        in_specs=[_row_spec(), _row_spec(), _row_spec(),
                  _bias_spec(), _full_spec()],
        out_specs=_row_spec(),
        out_shape=jax.ShapeDtypeStruct((NP, D), jnp.float32),
    )(a, u, dinvb, b, w)


def _final_body(a_ref, u_ref, dinv_ref, b_ref, o_ref):
    s = a_ref[...] + u_ref[...]
    o_ref[...] = jax.nn.relu(dinv_ref[...] * s + b_ref[...])


def _tc_final(a, u, dinvb, b):
    return pl.pallas_call(
        _final_body,
        grid=(_NBLK,),
        in_specs=[_row_spec(), _row_spec(), _row_spec(), _bias_spec()],
        out_specs=_row_spec(),
        out_shape=jax.ShapeDtypeStruct((NP, D), jnp.float32),
    )(a, u, dinvb, b)


def kernel(features, adj, W1, b1, W2, b2):
    adj = adj.astype(jnp.int32)
    src = jnp.concatenate([adj[0], jnp.zeros((EP - E,), jnp.int32)])
    dst = jnp.concatenate([adj[1], jnp.full((EP - E,), NP - 1, jnp.int32)])
    src3 = src.reshape(NW, NCHUNK, CHUNK)
    dst3 = dst.reshape(NW, NCHUNK, CHUNK)
    srcA = src.reshape(NS, NCHUNK_A, CHUNK)
    dstA = dst.reshape(NS, NCHUNK_A, CHUNK)

    featp = jnp.pad(features, ((0, NP - N), (0, 0)))
    zerosD = jnp.zeros((CHUNK, D), jnp.float32)
    b1r = b1.reshape(1, D)
    b2r = b2.reshape(1, D)

    degp = _sc_degree(dst3)
    u1, dinvb = _tc_mm_scale(featp, W1, degp)

    acc1 = _sc_aggregate(u1, srcA, dstA, zerosD)
    u2 = _tc_combine_mm(acc1, u1, dinvb, b1r, W2)

    acc2 = _sc_aggregate(u2, srcA, dstA, zerosD)
    h = _tc_final(acc2, u2, dinvb, b2r)

    return h[:N]


# speedup vs baseline: 8.8884x; 1.0032x over previous
import dataclasses
import functools

import jax
import jax.numpy as jnp
from jax import lax
from jax.experimental import pallas as pl
from jax.experimental.pallas import tpu as pltpu
from jax.experimental.pallas import tpu_sc as plsc

N = 10000
E = 320000
D = 128

NC = 2
NS = 16
NW = NC * NS
CHUNK = 128
NCHUNK = 80
STRIPE = 16
NSTRIPE = NCHUNK // STRIPE
EPW = CHUNK * NCHUNK
EP = EPW * NW
NCHUNK_A = 160
NP = 10240
RPT = NP // NS


@functools.cache
def _sc_degree_kernel():
    mesh = plsc.VectorSubcoreMesh(core_axis_name="c", subcore_axis_name="s")
    cp = pltpu.CompilerParams()
    if "needs_layout_passes" in pltpu.CompilerParams.__dataclass_fields__:
        cp = dataclasses.replace(cp, needs_layout_passes=False)
    return pl.kernel(
        _sc_degree_body,
        out_type=jax.ShapeDtypeStruct((NW, NP), jnp.float32),
        mesh=mesh,
        compiler_params=cp,
        scratch_types=[
            pltpu.VMEM((NCHUNK, CHUNK), jnp.int32),
            pltpu.VMEM((NP,), jnp.float32),
        ],
    )


def _sc_degree(dst3):
    return _sc_degree_kernel()(dst3)


def _sc_degree_body(dst_hbm, out_hbm, dstv, degv):
    c = lax.axis_index("c")
    s = lax.axis_index("s")
    w = c * NS + s
    pltpu.sync_copy(dst_hbm.at[w], dstv)

    @pl.loop(0, NP, step=16)
    def _(i):
        degv[pl.ds(i, 16)] = jnp.zeros((16,), jnp.float32)

    ones = jnp.full((16,), 1.0, jnp.float32)

    @pl.loop(0, NCHUNK)
    def _(j):
        for k in range(CHUNK // 16):
            iv = dstv[j, pl.ds(k * 16, 16)]
            plsc.addupdate_scatter(degv, [iv], ones)

    pltpu.sync_copy(degv, out_hbm.at[w])


@functools.cache
def _sc_aggregate_kernel():
    mesh = plsc.VectorSubcoreMesh(core_axis_name="c", subcore_axis_name="s")
    return pl.kernel(
        _sc_aggregate_body,
        out_type=jax.ShapeDtypeStruct((NP, D), jnp.float32),
        mesh=mesh,
        scratch_types=[
            pltpu.VMEM((STRIPE, CHUNK), jnp.int32),
            pltpu.VMEM((STRIPE, CHUNK), jnp.int32),
            pltpu.VMEM((CHUNK, D), jnp.float32),
            pltpu.VMEM((CHUNK, D), jnp.float32),
            pltpu.SemaphoreType.DMA,
            pltpu.SemaphoreType.DMA,
            pltpu.VMEM_SHARED((NP, D), jnp.float32),
        ],
    )


def _sc_aggregate(u, srcA, dstA, zerosD):
    return _sc_aggregate_kernel()(u, srcA, dstA, zerosD)


def _sc_aggregate_body(u_hbm, srcA_hbm, dstA_hbm,
                       zeros_hbm, out_hbm,
                       srcv, dstv, rows0, rows1, sem0, sem1, acc_sh):
    c = lax.axis_index("c")
    s = lax.axis_index("s")

    @pl.when(c == 0)
    def _():
        pltpu.sync_copy(zeros_hbm, rows0)

        @pl.loop(0, RPT // CHUNK)
        def _(k):
            pltpu.sync_copy(rows0, acc_sh.at[pl.ds(s * RPT + k * CHUNK, CHUNK)])

    plsc.subcore_barrier()

    def stripe(src_hbm, dst_hbm, t):
        pltpu.sync_copy(src_hbm.at[s, pl.ds(t * STRIPE, STRIPE)], srcv)
        pltpu.sync_copy(dst_hbm.at[s, pl.ds(t * STRIPE, STRIPE)], dstv)
        pltpu.async_copy(u_hbm.at[srcv.at[0]], rows0, sem0)

        @pl.loop(0, STRIPE // 2)
        def _(i):
            j = 2 * i
            h1 = pltpu.async_copy(u_hbm.at[srcv.at[j + 1]], rows1, sem1)
            pltpu.make_async_copy(u_hbm.at[srcv.at[j]], rows0, sem0).wait()
            pltpu.sync_copy(rows0, acc_sh.at[dstv.at[j]], add=True)

            @pl.when(j + 2 < STRIPE)
            def _():
                pltpu.async_copy(u_hbm.at[srcv.at[j + 2]], rows0, sem0)

            h1.wait()
            pltpu.sync_copy(rows1, acc_sh.at[dstv.at[j + 1]], add=True)

    @pl.when(c == 0)
    def _():
        @pl.loop(0, NCHUNK_A // STRIPE)
        def _(t):
            stripe(srcA_hbm, dstA_hbm, t)

    plsc.subcore_barrier()

    @pl.when(c == 0)
    def _():
        pltpu.sync_copy(acc_sh.at[pl.ds(s * RPT, RPT)],
                        out_hbm.at[pl.ds(s * RPT, RPT)])


_BLK = 1024
_NBLK = NP // _BLK


def _row_spec():
    return pl.BlockSpec((_BLK, D), lambda i: (i, 0))


def _full_spec():
    return pl.BlockSpec((D, D), lambda i: (0, 0))


def _bias_spec():
    return pl.BlockSpec((1, D), lambda i: (0, 0))


def _mm_scale_body(x_ref, w_ref, dp_ref, u_ref, dinv_ref):
    ones_w = jnp.ones((NW, 1), jnp.float32)
    colsum = lax.dot_general(dp_ref[...], ones_w, (((0,), (0,)), ((), ())),
                             preferred_element_type=jnp.float32)
    dinv = lax.rsqrt(1.0 + colsum)
    dinvb = jnp.broadcast_to(dinv, (_BLK, D))
    dinv_ref[...] = dinvb
    xw = jnp.dot(x_ref[...], w_ref[...], preferred_element_type=jnp.float32)
    u_ref[...] = dinvb * xw


def _tc_mm_scale(x, w, dp):
    deg_spec = pl.BlockSpec((NW, _BLK), lambda i: (0, i))
    return pl.pallas_call(
        _mm_scale_body,
        grid=(_NBLK,),
        in_specs=[_row_spec(), _full_spec(), deg_spec],
        out_specs=[_row_spec(), _row_spec()],
        out_shape=[jax.ShapeDtypeStruct((NP, D), jnp.float32),
                   jax.ShapeDtypeStruct((NP, D), jnp.float32)],
    )(x, w, dp)


def _combine_body(a_ref, u_ref, dinv_ref, b_ref, w_ref, o_ref):
    s = a_ref[...] + u_ref[...]
    h = jax.nn.relu(dinv_ref[...] * s + b_ref[...])
    o_ref[...] = dinv_ref[...] * jnp.dot(h, w_ref[...],
                                         preferred_element_type=jnp.float32)


def _tc_combine_mm(a, u, dinvb, b, w):
    return pl.pallas_call(
        _combine_body,
        grid=(_NBLK,),
        in_specs=[_row_spec(), _row_spec(), _row_spec(),
                  _bias_spec(), _full_spec()],
        out_specs=_row_spec(),
        out_shape=jax.ShapeDtypeStruct((NP, D), jnp.float32),
    )(a, u, dinvb, b, w)


def _final_body(a_ref, u_ref, dinv_ref, b_ref, o_ref):
    s = a_ref[...] + u_ref[...]
    o_ref[...] = jax.nn.relu(dinv_ref[...] * s + b_ref[...])


def _tc_final(a, u, dinvb, b):
    return pl.pallas_call(
        _final_body,
        grid=(_NBLK,),
        in_specs=[_row_spec(), _row_spec(), _row_spec(), _bias_spec()],
        out_specs=_row_spec(),
        out_shape=jax.ShapeDtypeStruct((NP, D), jnp.float32),
    )(a, u, dinvb, b)


def kernel(features, adj, W1, b1, W2, b2):
    adj = adj.astype(jnp.int32)
    src = jnp.concatenate([adj[0], jnp.zeros((EP - E,), jnp.int32)])
    dst = jnp.concatenate([adj[1], jnp.full((EP - E,), NP - 1, jnp.int32)])
    src3 = src.reshape(NW, NCHUNK, CHUNK)
    dst3 = dst.reshape(NW, NCHUNK, CHUNK)
    srcA = src.reshape(NS, NCHUNK_A, CHUNK)
    dstA = dst.reshape(NS, NCHUNK_A, CHUNK)

    featp = jnp.pad(features, ((0, NP - N), (0, 0)))
    zerosD = jnp.zeros((CHUNK, D), jnp.float32)
    b1r = b1.reshape(1, D)
    b2r = b2.reshape(1, D)

    degp = _sc_degree(dst3)
    u1, dinvb = _tc_mm_scale(featp, W1, degp)

    acc1 = _sc_aggregate(u1, srcA, dstA, zerosD)
    u2 = _tc_combine_mm(acc1, u1, dinvb, b1r, W2)

    acc2 = _sc_aggregate(u2, srcA, dstA, zerosD)
    h = _tc_final(acc2, u2, dinvb, b2r)

    return h[:N]
